# baseline (speedup 1.0000x reference)
import functools

import jax
import jax.numpy as jnp
from jax import lax
from jax.experimental import pallas as pl
from jax.experimental.pallas import tpu as pltpu
from jax.experimental.pallas import tpu_sc as plsc

_B, _N, _C = 64, 8192, 16
_L = 16
_V = _N // _L
_NT = _N // 128
_NBINS = 256
_NPASS = 4
_ROWS_PER_W = _B // 32
_MIN32 = -2147483648


def _body(x6_hbm, out6_hbm, inA, inB, keyA, keyB, payA, payB, off, pidx,
          ob0, ob1, semi, sem0, sem1):
    iota = lax.iota(jnp.int32, _L)
    ones = jnp.ones((_L,), jnp.int32)
    zeros = jnp.zeros((_L,), jnp.int32)
    cvecs = [jnp.full((_L,), c, jnp.int32) for c in range(4)]
    wid = lax.axis_index("s") * 2 + lax.axis_index("c")

    def src(b, ct, ch):
        return x6_hbm.at[b, ct, :, pl.ds(ch * 4, 4), :]

    def do_row(r, _):
        b = wid * _ROWS_PER_W + r

        pltpu.sync_copy(src(b, 0, 0), inA)

        @plsc.parallel_loop(0, _V, unroll=4)
        def init_body(v):
            kf = inA[v >> 3, 0, pl.ds((v & 7) * _L, _L)]
            k = plsc.bitcast(kf, jnp.int32)
            k = jnp.where(k == _MIN32, 0, k)
            t = k ^ _MIN32
            d = jnp.where(k >= 0, ~t, k)
            addr = (v & 31) * 256 + (v >> 5) + iota * _L
            plsc.store_scatter(keyA, [addr], d)
            payA[pl.ds(v * _L, _L)] = iota * _V + v

        pltpu.async_copy(src(b, 0, 1), inB, semi)

        for p in range(_NPASS):
            shift = jnp.full((_L,), 8 * p, jnp.int32)
            ik, ip, ok_, op_ = ((keyA, payA, keyB, payB) if p % 2 == 0
                                else (keyB, payB, keyA, payA))

            @plsc.parallel_loop(0, _NBINS, unroll=4)
            def zero_body(bb):
                off[pl.ds(bb * _L, _L)] = zeros

            @plsc.parallel_loop(0, _V, unroll=8)
            def hist_body(v):
                d = ik[pl.ds(v * _L, _L)]
                dig = lax.shift_right_logical(d, shift) & 255
                plsc.addupdate_scatter(off, [dig * _L + iota], ones)

            @plsc.parallel_loop(0, _NBINS, unroll=2, carry=jnp.int32(0))
            def scan_body(bb, carry):
                h = off[pl.ds(bb * _L, _L)]
                cs = plsc.cumsum(h)
                off[pl.ds(bb * _L, _L)] = cs - h + carry
                return carry + jnp.sum(h)

            def perm_body(v8, _):
                ds_, pvs, digs, addrs = [], [], [], []
                for k in range(8):
                    v = v8 * 8 + k
                    ds_.append(ik[pl.ds(v * _L, _L)])
                    pvs.append(ip[pl.ds(v * _L, _L)])
                    digs.append(lax.shift_right_logical(ds_[k], shift) & 255)
                    addrs.append(digs[k] * _L + iota)
                ss = [plsc.load_gather(off, [a_]) for a_ in addrs]
                for k in range(8):
                    for j in range(k):
                        ss[k] = ss[k] + jnp.where(digs[k] == digs[j], 1, 0)
                for k in range(8):
                    plsc.addupdate_scatter(off, [addrs[k]], ones)
                for k in range(8):
                    s = ss[k]
                    a = ((s & 511) << 4) | (s >> 9)
                    if p < _NPASS - 1:
                        plsc.store_scatter(ok_, [a], ds_[k])
                    plsc.store_scatter(op_, [a], pvs[k])
            lax.fori_loop(0, _V // 8, perm_body, None)

        @plsc.parallel_loop(0, _V, unroll=4)
        def untr_body(w):
            base = (w & 31) * 256 + (w >> 5)
            g = plsc.load_gather(payA, [base + iota * _L])
            pidx[pl.ds(w * _L, _L)] = g

        def gather_chunk(nt8, blk, ob):
            @plsc.parallel_loop(0, 8, unroll=4)
            def _(u):
                sbase = nt8 * 1024 + u * 128
                for vv in range(8):
                    bases = pidx[pl.ds(sbase + vv * _L, _L)]
                    i_nt = bases >> 7
                    i_nl = bases & 127
                    for c in range(4):
                        g = plsc.load_gather(blk, [i_nt, cvecs[c], i_nl])
                        ob[u, c, pl.ds(vv * _L, _L)] = g

        def gather_half(ct, ch, blk):
            def dst(nt8):
                return out6_hbm.at[b, ct, pl.ds(nt8 * 8, 8),
                                   pl.ds(ch * 4, 4), :]

            def g_body(cc, _):
                nt8 = cc * 2

                @pl.when(cc > 0)
                def _():
                    pltpu.make_async_copy(ob1, dst(nt8 - 1), sem1).wait()
                gather_chunk(nt8, blk, ob0)
                pltpu.async_copy(ob0, dst(nt8), sem0)
                gather_chunk(nt8 + 1, blk, ob1)
                pltpu.make_async_copy(ob0, dst(nt8), sem0).wait()
                pltpu.async_copy(ob1, dst(nt8 + 1), sem1)
            lax.fori_loop(0, 4, g_body, None)
            pltpu.make_async_copy(ob1, dst(7), sem1).wait()

        gather_half(0, 0, inA)
        pltpu.make_async_copy(src(b, 0, 1), inB, semi).wait()
        pltpu.async_copy(src(b, 1, 0), inA, semi)
        gather_half(0, 1, inB)
        pltpu.make_async_copy(src(b, 1, 0), inA, semi).wait()
        pltpu.async_copy(src(b, 1, 1), inB, semi)
        gather_half(1, 0, inA)
        pltpu.make_async_copy(src(b, 1, 1), inB, semi).wait()
        gather_half(1, 1, inB)

    lax.fori_loop(0, _ROWS_PER_W, do_row, None)


_sc_sort = functools.partial(
    pl.kernel,
    out_type=jax.ShapeDtypeStruct((_B, 2, _NT, 8, 128), jnp.float32),
    mesh=plsc.VectorSubcoreMesh(core_axis_name="c", subcore_axis_name="s",
                                num_cores=2, num_subcores=16),
    scratch_types=[
        pltpu.VMEM((_NT, 4, 128), jnp.float32),
        pltpu.VMEM((_NT, 4, 128), jnp.float32),
        pltpu.VMEM((_N,), jnp.int32),
        pltpu.VMEM((_N,), jnp.int32),
        pltpu.VMEM((_N,), jnp.int32),
        pltpu.VMEM((_N,), jnp.int32),
        pltpu.VMEM((_NBINS * _L,), jnp.int32),
        pltpu.VMEM((_N,), jnp.int32),
        pltpu.VMEM((8, 4, 128), jnp.float32),
        pltpu.VMEM((8, 4, 128), jnp.float32),
        pltpu.SemaphoreType.DMA,
        pltpu.SemaphoreType.DMA,
        pltpu.SemaphoreType.DMA,
    ],
    compiler_params=pltpu.CompilerParams(needs_layout_passes=False,
                                         use_tc_tiling_on_sc=False),
)(_body)


@jax.jit
def kernel(x):
    xt = lax.transpose(x, (0, 2, 1))
    x6 = xt.reshape(_B, 2, 8, _NT, 128).transpose(0, 1, 3, 2, 4)
    o6 = _sc_sort(x6)
    out = o6.transpose(0, 1, 3, 2, 4).reshape(_B, _C, _N)
    return lax.transpose(out, (0, 2, 1))
